# Initial kernel scaffold; baseline (speedup 1.0000x reference)
import jax
import jax.numpy as jnp
from jax.experimental import pallas as pl


def kernel(edge_index, edge_weight, user_emb, item_emb):
    raise NotImplementedError("write your pallas kernel here")


# speedup vs baseline: 2.0074x; 2.0074x over previous
import functools

import jax
import jax.numpy as jnp
from jax import lax
from jax.experimental import pallas as pl
from jax.experimental.pallas import tpu as pltpu
from jax.experimental.pallas import tpu_sc as plsc

NU = 50000
NI = 50000
D = 64
NE = 800000
NLAYERS = 3

CHUNK = 128
SUP = 32
SUPW = 16
NEP = 851968
EROWS = NEP // CHUNK
PAD_NODE = 50000
NPAD = 50176
HALF = 25000
DUMP = 25000
ACC_ROWS = 25216

_MESH = plsc.VectorSubcoreMesh(core_axis_name="c", subcore_axis_name="s")


def _rsqrt16(x):
    i = plsc.bitcast(x, jnp.int32)
    i = jnp.int32(0x5F3759DF) - lax.shift_right_logical(i, 1)
    y = plsc.bitcast(i, jnp.float32)
    for _ in range(3):
        y = y * (1.5 - 0.5 * x * y * y)
    return y


@functools.lru_cache(maxsize=None)
def _make_w_kernel():
    @functools.partial(
        pl.kernel,
        out_type=jax.ShapeDtypeStruct((EROWS, CHUNK), jnp.float32),
        mesh=_MESH,
        compiler_params=pltpu.CompilerParams(needs_layout_passes=False, use_tc_tiling_on_sc=False),
        scratch_types=[
            pltpu.VMEM((SUP, CHUNK), jnp.int32),
            pltpu.VMEM((SUP, CHUNK), jnp.int32),
            pltpu.VMEM((SUP, CHUNK), jnp.float32),
            pltpu.VMEM((SUP, CHUNK), jnp.float32),
            pltpu.VMEM((CHUNK,), jnp.float32),
            pltpu.VMEM((3136,), jnp.float32),
            pltpu.VMEM((NPAD,), jnp.float32),
            pltpu.VMEM((NPAD,), jnp.float32),
            pltpu.VMEM_SHARED((NPAD,), jnp.float32),
            pltpu.VMEM_SHARED((NPAD,), jnp.float32),
            pltpu.SemaphoreType.DMA,
            pltpu.SemaphoreType.DMA,
        ],
    )
    def w_kernel(src_hbm, dst_hbm, ew_hbm, w_hbm,
                 s2d, d2d, e2d, wout, ones, zbuf, degu_v, degi_v,
                 degu_sp, degi_sp, usem, isem):
        cid = lax.axis_index("c")
        sid = lax.axis_index("s")

        def z16(i, _):
            zbuf[pl.ds(i * 16, 16)] = jnp.zeros((16,), jnp.float32)
            return 0
        lax.fori_loop(0, 3136 // 16, z16, 0)
        for q in range(CHUNK // 16):
            ones[pl.ds(q * 16, 16)] = jnp.ones((16,), jnp.float32)

        pltpu.sync_copy(zbuf, degu_sp.at[pl.ds(sid * 3136, 3136)])
        pltpu.sync_copy(zbuf, degi_sp.at[pl.ds(sid * 3136, 3136)])
        plsc.subcore_barrier()

        row0 = sid * (EROWS // 16)
        def sup_deg(s, _):
            base = row0 + s * SUP
            pltpu.sync_copy(src_hbm.at[pl.ds(base, SUP)], s2d)
            pltpu.sync_copy(dst_hbm.at[pl.ds(base, SUP)], d2d)
            def fire(c, _):
                pltpu.async_copy(ones, degu_sp.at[s2d.at[c]], usem, add=True)
                pltpu.async_copy(ones, degi_sp.at[d2d.at[c]], isem, add=True)
                return 0
            lax.fori_loop(0, SUP, fire, 0)
            def drain(c, _):
                pltpu.make_async_copy(ones, degu_sp.at[s2d.at[c]], usem).wait()
                pltpu.make_async_copy(ones, degi_sp.at[d2d.at[c]], isem).wait()
                return 0
            lax.fori_loop(0, SUP, drain, 0)
            return 0
        lax.fori_loop(0, (EROWS // 16) // SUP, sup_deg, 0)
        plsc.subcore_barrier()

        pltpu.sync_copy(degu_sp, degu_v)
        pltpu.sync_copy(degi_sp, degi_v)

        wid = cid * 16 + sid
        myrow0 = wid * (EROWS // 32)
        def sup_w(s, _):
            base = myrow0 + s * SUPW
            pltpu.sync_copy(src_hbm.at[pl.ds(base, SUPW)],
                            s2d.at[pl.ds(0, SUPW)])
            pltpu.sync_copy(dst_hbm.at[pl.ds(base, SUPW)],
                            d2d.at[pl.ds(0, SUPW)])
            pltpu.sync_copy(ew_hbm.at[pl.ds(base, SUPW)],
                            e2d.at[pl.ds(0, SUPW)])
            def crow(c, _):
                def k16(k, _):
                    sl = pl.ds(k * 16, 16)
                    du = plsc.load_gather(degu_v, [s2d[c, sl]])
                    di = plsc.load_gather(degi_v, [d2d[c, sl]])
                    x = jnp.maximum(du, 1.0) * jnp.maximum(di, 1.0)
                    wout[c, sl] = e2d[c, sl] * _rsqrt16(x)
                    return 0
                lax.fori_loop(0, CHUNK // 16, k16, 0)
                return 0
            lax.fori_loop(0, SUPW, crow, 0)
            pltpu.sync_copy(wout.at[pl.ds(0, SUPW)], w_hbm.at[pl.ds(base, SUPW)])
            return 0
        lax.fori_loop(0, (EROWS // 32) // SUPW, sup_w, 0)

    return w_kernel


@functools.lru_cache(maxsize=None)
def _make_prop_kernel():
    @functools.partial(
        pl.kernel,
        out_type=jax.ShapeDtypeStruct((NPAD, D), jnp.float32),
        mesh=_MESH,
        compiler_params=pltpu.CompilerParams(needs_layout_passes=False, use_tc_tiling_on_sc=False),
        scratch_types=[
            pltpu.VMEM((SUP, CHUNK), jnp.int32),
            pltpu.VMEM((SUP, CHUNK), jnp.int32),
            pltpu.VMEM((SUP, CHUNK), jnp.float32),
            pltpu.VMEM((CHUNK, D), jnp.float32),
            pltpu.VMEM_SHARED((ACC_ROWS, D), jnp.float32),
            pltpu.SemaphoreType.DMA,
            pltpu.SemaphoreType.DMA,
        ],
    )
    def prop_kernel(x_hbm, g_hbm, t_hbm, w_hbm, out_hbm,
                    g2d, t2d, w2d, rows, acc, gsem, ssem):
        cid = lax.axis_index("c")
        sid = lax.axis_index("s")
        nbase = cid * HALF

        def zr(r, _):
            for q in range(D // 16):
                rows[r, pl.ds(q * 16, 16)] = jnp.zeros((16,), jnp.float32)
            return 0
        lax.fori_loop(0, CHUNK, zr, 0)
        a0 = sid * (ACC_ROWS // 16)
        def zc(i, _):
            pltpu.sync_copy(rows, acc.at[pl.ds(a0 + i * CHUNK, CHUNK)])
            return 0
        lax.fori_loop(0, 12, zc, 0)
        pltpu.sync_copy(rows.at[pl.ds(0, 40)],
                        acc.at[pl.ds(a0 + 12 * CHUNK, 40)])
        plsc.subcore_barrier()

        row0 = sid * (EROWS // 16)
        def sup(s, _):
            base = row0 + s * SUP
            pltpu.sync_copy(g_hbm.at[pl.ds(base, SUP)], g2d)
            pltpu.sync_copy(t_hbm.at[pl.ds(base, SUP)], t2d)
            pltpu.sync_copy(w_hbm.at[pl.ds(base, SUP)], w2d)
            def rc(c, _):
                def rk(k, _):
                    sl = pl.ds(k * 16, 16)
                    t16 = t2d[c, sl] - nbase
                    bad = (t16 < 0) | (t16 >= HALF)
                    t2d[c, sl] = jnp.where(
                        bad, jnp.full((16,), DUMP, jnp.int32), t16)
                    return 0
                lax.fori_loop(0, CHUNK // 16, rk, 0)
                return 0
            lax.fori_loop(0, SUP, rc, 0)

            def ch(c, _):
                pltpu.async_copy(x_hbm.at[g2d.at[c]], rows, gsem).wait()
                def sr(r, _):
                    wb = plsc.load_gather(
                        w2d, [jnp.full((16,), c, jnp.int32),
                              jnp.full((16,), r, jnp.int32)])
                    for q in range(D // 16):
                        sl = pl.ds(q * 16, 16)
                        rows[r, sl] = rows[r, sl] * wb
                    return 0
                lax.fori_loop(0, CHUNK, sr, 0)
                pltpu.async_copy(rows, acc.at[t2d.at[c]], ssem, add=True).wait()
                return 0
            lax.fori_loop(0, SUP, ch, 0)
            return 0
        lax.fori_loop(0, (EROWS // 16) // SUP, sup, 0)
        plsc.subcore_barrier()

        @pl.when(cid == 0)
        def _():
            pltpu.sync_copy(acc.at[pl.ds(sid * 1560, 1560)],
                            out_hbm.at[pl.ds(sid * 1560, 1560)])
        @pl.when((cid == 0) & (sid == 0))
        def _():
            pltpu.sync_copy(acc.at[pl.ds(24960, 40)],
                            out_hbm.at[pl.ds(24960, 40)])
        @pl.when(cid == 1)
        def _():
            pltpu.sync_copy(acc.at[pl.ds(sid * 1568, 1568)],
                            out_hbm.at[pl.ds(HALF + sid * 1568, 1568)])
        @pl.when((cid == 1) & (sid == 0))
        def _():
            pltpu.sync_copy(acc.at[pl.ds(25088, 88)],
                            out_hbm.at[pl.ds(50088, 88)])

    return prop_kernel


@functools.lru_cache(maxsize=None)
def _make_mean_kernel():
    @functools.partial(
        pl.kernel,
        out_type=jax.ShapeDtypeStruct((NU, D), jnp.float32),
        mesh=_MESH,
        compiler_params=pltpu.CompilerParams(needs_layout_passes=False, use_tc_tiling_on_sc=False),
        scratch_types=[
            pltpu.VMEM((CHUNK, D), jnp.float32),
            pltpu.VMEM((CHUNK, D), jnp.float32),
            pltpu.VMEM((CHUNK, D), jnp.float32),
            pltpu.VMEM((CHUNK, D), jnp.float32),
            pltpu.VMEM((CHUNK, D), jnp.float32),
        ],
    )
    def mean_kernel(e0, e1, e2, e3, out, b0, b1, b2, b3, ob):
        cid = lax.axis_index("c")
        sid = lax.axis_index("s")
        wid = cid * 16 + sid
        base = wid * 1560

        def do_chunk(off, n):
            pltpu.sync_copy(e0.at[pl.ds(off, n)], b0.at[pl.ds(0, n)])
            pltpu.sync_copy(e1.at[pl.ds(off, n)], b1.at[pl.ds(0, n)])
            pltpu.sync_copy(e2.at[pl.ds(off, n)], b2.at[pl.ds(0, n)])
            pltpu.sync_copy(e3.at[pl.ds(off, n)], b3.at[pl.ds(0, n)])
            def rr(r, _):
                for q in range(D // 16):
                    sl = pl.ds(q * 16, 16)
                    ob[r, sl] = (b0[r, sl] + b1[r, sl]
                                 + b2[r, sl] + b3[r, sl]) * 0.25
                return 0
            lax.fori_loop(0, n, rr, 0)
            pltpu.sync_copy(ob.at[pl.ds(0, n)], out.at[pl.ds(off, n)])

        def loop(i, _):
            do_chunk(base + i * CHUNK, CHUNK)
            return 0
        lax.fori_loop(0, 12, loop, 0)
        do_chunk(base + 12 * CHUNK, 24)
        @pl.when(wid == 0)
        def _():
            do_chunk(49920, 80)

    return mean_kernel


def kernel(edge_index, edge_weight, user_emb, item_emb):
    src = edge_index[0].astype(jnp.int32)
    dst = edge_index[1].astype(jnp.int32)
    ew = edge_weight.astype(jnp.float32)

    padn = jnp.full((NEP - NE,), PAD_NODE, jnp.int32)
    s2 = jnp.concatenate([src, padn]).reshape(EROWS, CHUNK)
    d2 = jnp.concatenate([dst, padn]).reshape(EROWS, CHUNK)
    e2 = jnp.concatenate(
        [ew, jnp.zeros((NEP - NE,), jnp.float32)]).reshape(EROWS, CHUNK)
    zrows = jnp.zeros((NPAD - NU, D), jnp.float32)
    xu = jnp.concatenate([user_emb.astype(jnp.float32), zrows])
    xi = jnp.concatenate([item_emb.astype(jnp.float32), zrows])

    w2 = _make_w_kernel()(s2, d2, e2)

    prop = _make_prop_kernel()
    u, i = xu, xi
    us, its = [], []
    for _ in range(NLAYERS):
        u_new = prop(i, d2, s2, w2)
        i_new = prop(u, s2, d2, w2)
        u, i = u_new, i_new
        us.append(u)
        its.append(i)

    mean = _make_mean_kernel()
    fu = mean(user_emb, us[0], us[1], us[2])
    fi = mean(item_emb, its[0], its[1], its[2])
    return (fu, fi)
